# Initial kernel scaffold; baseline (speedup 1.0000x reference)
import functools

import jax
import jax.numpy as jnp
from jax import lax
from jax.experimental import pallas as pl
from jax.experimental.pallas import tpu as pltpu
from jax.experimental.pallas import tpu_sc as plsc

B = 16384
D_ARTIST, D_ALBUM, D_GENRE = 64, 32, 16
H1, H2 = 256, 128

NC, NS = 2, 16
NW = NC * NS
BPW = B // NW
NCHUNK = BPW // 128

_sc_mesh = plsc.VectorSubcoreMesh(core_axis_name="c", subcore_axis_name="s")


@functools.partial(
    pl.kernel,
    out_type=[
        jax.ShapeDtypeStruct((B, D_ARTIST), jnp.float32),
        jax.ShapeDtypeStruct((B, D_ALBUM), jnp.float32),
        jax.ShapeDtypeStruct((B, D_GENRE), jnp.float32),
    ],
    mesh=_sc_mesh,
    scratch_types=[
        pltpu.VMEM((NCHUNK, 128), jnp.int32),
        pltpu.VMEM((NCHUNK, 128), jnp.int32),
        pltpu.VMEM((NCHUNK, 128), jnp.int32),
        pltpu.VMEM((BPW, D_ARTIST), jnp.float32),
        pltpu.VMEM((BPW, D_ALBUM), jnp.float32),
        pltpu.VMEM((BPW, D_GENRE), jnp.float32),
        pltpu.SemaphoreType.DMA,
    ],
)
def _sc_gather(aid_h, bid_h, gid_h, ea_h, eb_h, eg_h,
               oa_h, ob_h, og_h,
               ia_v, ib_v, ig_v, ra_v, rb_v, rg_v, sem):
    wid = lax.axis_index("s") * NC + lax.axis_index("c")
    row0 = wid * NCHUNK
    base = wid * BPW
    pltpu.sync_copy(aid_h.at[pl.ds(row0, NCHUNK)], ia_v)
    pltpu.sync_copy(bid_h.at[pl.ds(row0, NCHUNK)], ib_v)
    pltpu.sync_copy(gid_h.at[pl.ds(row0, NCHUNK)], ig_v)
    copies = []
    for j in range(NCHUNK):
        sl = pl.ds(j * 128, 128)
        copies.append(pltpu.async_copy(ea_h.at[ia_v.at[j]], ra_v.at[sl], sem))
        copies.append(pltpu.async_copy(eb_h.at[ib_v.at[j]], rb_v.at[sl], sem))
        copies.append(pltpu.async_copy(eg_h.at[ig_v.at[j]], rg_v.at[sl], sem))
    for c in copies:
        c.wait()
    pltpu.sync_copy(ra_v, oa_h.at[pl.ds(base, BPW)])
    pltpu.sync_copy(rb_v, ob_h.at[pl.ds(base, BPW)])
    pltpu.sync_copy(rg_v, og_h.at[pl.ds(base, BPW)])


BLK = 2048


def _mlp_body(num_ref, ea_ref, eb_ref, eg_ref,
              w1n_ref, w1a_ref, w1b_ref, w1g_ref, b1_ref, w2_ref, b2_ref,
              o_ref):
    h = jnp.dot(num_ref[...], w1n_ref[...], preferred_element_type=jnp.float32)
    h += jnp.dot(ea_ref[...], w1a_ref[...], preferred_element_type=jnp.float32)
    h += jnp.dot(eb_ref[...], w1b_ref[...], preferred_element_type=jnp.float32)
    h += jnp.dot(eg_ref[...], w1g_ref[...], preferred_element_type=jnp.float32)
    h = jnp.maximum(h + b1_ref[...], 0.0)
    o = jnp.dot(h, w2_ref[...], preferred_element_type=jnp.float32)
    o_ref[...] = jnp.maximum(o + b2_ref[...], 0.0)


def _mlp(num, ea, eb, eg, w1n, w1a, w1b, w1g, b1, w2, b2):
    grid = (B // BLK,)
    full = lambda shape: pl.BlockSpec(shape, lambda i: (0, 0))
    return pl.pallas_call(
        _mlp_body,
        grid=grid,
        in_specs=[
            pl.BlockSpec((BLK, 9), lambda i: (i, 0)),
            pl.BlockSpec((BLK, D_ARTIST), lambda i: (i, 0)),
            pl.BlockSpec((BLK, D_ALBUM), lambda i: (i, 0)),
            pl.BlockSpec((BLK, D_GENRE), lambda i: (i, 0)),
            full((9, H1)),
            full((D_ARTIST, H1)),
            full((D_ALBUM, H1)),
            full((D_GENRE, H1)),
            full((1, H1)),
            full((H1, H2)),
            full((1, H2)),
        ],
        out_specs=pl.BlockSpec((BLK, H2), lambda i: (i, 0)),
        out_shape=jax.ShapeDtypeStruct((B, H2), jnp.float32),
    )(num, ea, eb, eg, w1n, w1a, w1b, w1g, b1, w2, b2)


def kernel(danceability, energy, loudness, speechiness, acousticness,
           instrumentalness, liveness, valence, tempo,
           artist_id, album_id, genre_id,
           E_artist, E_album, E_genre, W1, b1, W2, b2):
    aid2 = artist_id.reshape(NW * NCHUNK, 128)
    bid2 = album_id.reshape(NW * NCHUNK, 128)
    gid2 = genre_id.reshape(NW * NCHUNK, 128)
    ea, eb, eg = _sc_gather(aid2, bid2, gid2, E_artist, E_album, E_genre)
    num = jnp.stack([danceability, energy, loudness, speechiness, acousticness,
                     instrumentalness, liveness, valence, tempo], axis=1)
    return _mlp(num, ea, eb, eg,
                W1[:9], W1[9:9 + D_ARTIST],
                W1[9 + D_ARTIST:9 + D_ARTIST + D_ALBUM],
                W1[9 + D_ARTIST + D_ALBUM:],
                b1.reshape(1, H1), W2, b2.reshape(1, H2))


# speedup vs baseline: 2.4888x; 2.4888x over previous
import functools

import jax
import jax.numpy as jnp
from jax import lax
from jax.experimental import pallas as pl
from jax.experimental.pallas import tpu as pltpu
from jax.experimental.pallas import tpu_sc as plsc

B = 16384
D_ARTIST, D_ALBUM, D_GENRE = 64, 32, 16
H1, H2 = 256, 128

NC, NS = 2, 16
NW = NC * NS
BPW = B // NW
NCHUNK = BPW // 128

_sc_mesh = plsc.VectorSubcoreMesh(core_axis_name="c", subcore_axis_name="s")


@functools.partial(
    pl.kernel,
    out_type=[
        jax.ShapeDtypeStruct((B, D_ARTIST), jnp.float32),
        jax.ShapeDtypeStruct((B, D_ALBUM), jnp.float32),
        jax.ShapeDtypeStruct((B, D_GENRE), jnp.float32),
    ],
    mesh=_sc_mesh,
    scratch_types=[
        pltpu.VMEM((NCHUNK, 128), jnp.int32),
        pltpu.VMEM((NCHUNK, 128), jnp.int32),
        pltpu.VMEM((NCHUNK, 128), jnp.int32),
        pltpu.VMEM((BPW, D_ARTIST), jnp.float32),
        pltpu.VMEM((BPW, D_ALBUM), jnp.float32),
        pltpu.VMEM((BPW, D_GENRE), jnp.float32),
        pltpu.SemaphoreType.DMA,
    ],
)
def _sc_gather(aid_h, bid_h, gid_h, ea_h, eb_h, eg_h,
               oa_h, ob_h, og_h,
               ia_v, ib_v, ig_v, ra_v, rb_v, rg_v, sem):
    wid = lax.axis_index("s") * NC + lax.axis_index("c")
    row0 = wid * NCHUNK
    base = wid * BPW
    pltpu.sync_copy(aid_h.at[pl.ds(row0, NCHUNK)], ia_v)
    pltpu.sync_copy(bid_h.at[pl.ds(row0, NCHUNK)], ib_v)
    pltpu.sync_copy(gid_h.at[pl.ds(row0, NCHUNK)], ig_v)
    copies = []
    for j in range(NCHUNK):
        sl = pl.ds(j * 128, 128)
        copies.append(pltpu.async_copy(ea_h.at[ia_v.at[j]], ra_v.at[sl], sem))
        copies.append(pltpu.async_copy(eb_h.at[ib_v.at[j]], rb_v.at[sl], sem))
        copies.append(pltpu.async_copy(eg_h.at[ig_v.at[j]], rg_v.at[sl], sem))
    for c in copies:
        c.wait()
    pltpu.sync_copy(ra_v, oa_h.at[pl.ds(base, BPW)])
    pltpu.sync_copy(rb_v, ob_h.at[pl.ds(base, BPW)])
    pltpu.sync_copy(rg_v, og_h.at[pl.ds(base, BPW)])


BLK = 2048


def _mlp_body(num_ref, ea_ref, eb_ref, eg_ref,
              w1n_ref, w1a_ref, w1b_ref, w1g_ref, b1_ref, w2_ref, b2_ref,
              o_ref):
    h = jnp.dot(num_ref[...], w1n_ref[...], preferred_element_type=jnp.float32)
    h += jnp.dot(ea_ref[...], w1a_ref[...], preferred_element_type=jnp.float32)
    h += jnp.dot(eb_ref[...], w1b_ref[...], preferred_element_type=jnp.float32)
    h += jnp.dot(eg_ref[...], w1g_ref[...], preferred_element_type=jnp.float32)
    h = jnp.maximum(h + b1_ref[...], 0.0)
    o = jnp.dot(h, w2_ref[...], preferred_element_type=jnp.float32)
    o_ref[...] = jnp.maximum(o + b2_ref[...], 0.0)


def _mlp(num, ea, eb, eg, w1n, w1a, w1b, w1g, b1, w2, b2):
    grid = (B // BLK,)
    full = lambda shape: pl.BlockSpec(shape, lambda i: (0, 0))
    return pl.pallas_call(
        _mlp_body,
        grid=grid,
        in_specs=[
            pl.BlockSpec((BLK, 9), lambda i: (i, 0)),
            pl.BlockSpec((BLK, D_ARTIST), lambda i: (i, 0)),
            pl.BlockSpec((BLK, D_ALBUM), lambda i: (i, 0)),
            pl.BlockSpec((BLK, D_GENRE), lambda i: (i, 0)),
            full((9, H1)),
            full((D_ARTIST, H1)),
            full((D_ALBUM, H1)),
            full((D_GENRE, H1)),
            full((1, H1)),
            full((H1, H2)),
            full((1, H2)),
        ],
        out_specs=pl.BlockSpec((BLK, H2), lambda i: (i, 0)),
        out_shape=jax.ShapeDtypeStruct((B, H2), jnp.float32),
    )(num, ea, eb, eg, w1n, w1a, w1b, w1g, b1, w2, b2)


def kernel(danceability, energy, loudness, speechiness, acousticness,
           instrumentalness, liveness, valence, tempo,
           artist_id, album_id, genre_id,
           E_artist, E_album, E_genre, W1, b1, W2, b2):
    ea = jnp.take(E_artist, artist_id, axis=0)
    eb = jnp.take(E_album, album_id, axis=0)
    eg = jnp.take(E_genre, genre_id, axis=0)
    num = jnp.stack([danceability, energy, loudness, speechiness, acousticness,
                     instrumentalness, liveness, valence, tempo], axis=1)
    return _mlp(num, ea, eb, eg,
                W1[:9], W1[9:9 + D_ARTIST],
                W1[9 + D_ARTIST:9 + D_ARTIST + D_ALBUM],
                W1[9 + D_ARTIST + D_ALBUM:],
                b1.reshape(1, H1), W2, b2.reshape(1, H2))
